# baseline (speedup 1.0000x reference)
import functools

import jax
import jax.numpy as jnp
from jax import lax
from jax.experimental import pallas as pl
from jax.experimental.pallas import tpu as pltpu
from jax.experimental.pallas import tpu_sc as plsc

_B = 4
_N = 2048
_S = 512
_K = 32
_D = 256
_ROWS = 8
_COLS = _N // _ROWS


def _fps_body(x_ref, y_ref, z_ref, idx_ref, qx_ref, qy_ref, qz_ref):
    X = x_ref[...]
    Y = y_ref[...]
    Z = z_ref[...]
    pos = (lax.broadcasted_iota(jnp.int32, (_B, _ROWS, _COLS), 1) * _COLS
           + lax.broadcasted_iota(jnp.int32, (_B, _ROWS, _COLS), 2))
    cols = lax.broadcasted_iota(jnp.int32, (_B, _S), 1)

    def combine(a, b):
        av, ai = a[0], a[1]
        bv, bi = b[0], b[1]
        take_a = (av > bv) | ((av == bv) & (ai < bi))
        return tuple(jnp.where(take_a, p, q) for p, q in zip(a, b))

    def winner(dists):
        t = (dists, pos, X, Y, Z)
        cur = combine(tuple(u[:, :, :_COLS // 2] for u in t),
                      tuple(u[:, :, _COLS // 2:] for u in t))
        for s in (1, 2, 4, 8, 16, 32, 64):
            cur = combine(cur, tuple(pltpu.roll(u, s, axis=2) for u in cur))
        for s in (1, 2, 4):
            cur = combine(cur, tuple(pltpu.roll(u, s, axis=1) for u in cur))
        return tuple(u[:, 0:1, 0:1] for u in cur)

    def body(i, carry):
        dists, fv, fi, cx, cy, cz, idx_acc, qxa, qya, qza = carry
        wm = cols == i
        idx_acc = jnp.where(wm, fi.reshape(_B, 1), idx_acc)
        qxa = jnp.where(wm, cx.reshape(_B, 1), qxa)
        qya = jnp.where(wm, cy.reshape(_B, 1), qya)
        qza = jnp.where(wm, cz.reshape(_B, 1), qza)
        dx = X - cx
        dy = Y - cy
        dz = Z - cz
        d = dx * dx + dy * dy + dz * dz
        dists = jnp.minimum(dists, d)
        fv, fi, cx, cy, cz = winner(dists)
        return dists, fv, fi, cx, cy, cz, idx_acc, qxa, qya, qza

    init = (jnp.full((_B, _ROWS, _COLS), 1e10, jnp.float32),
            jnp.zeros((_B, 1, 1), jnp.float32),
            jnp.zeros((_B, 1, 1), jnp.int32),
            X[:, 0:1, 0:1], Y[:, 0:1, 0:1], Z[:, 0:1, 0:1],
            jnp.zeros((_B, _S), jnp.int32),
            jnp.zeros((_B, _S), jnp.float32),
            jnp.zeros((_B, _S), jnp.float32),
            jnp.zeros((_B, _S), jnp.float32))
    out = lax.fori_loop(0, _S, body, init)
    _, _, _, _, _, _, idx_acc, qxa, qya, qza = out
    idx_ref[...] = idx_acc
    qx_ref[...] = qxa
    qy_ref[...] = qya
    qz_ref[...] = qza


def _fps(X, Y, Z):
    return pl.pallas_call(
        _fps_body,
        out_shape=(jax.ShapeDtypeStruct((_B, _S), jnp.int32),
                   jax.ShapeDtypeStruct((_B, _S), jnp.float32),
                   jax.ShapeDtypeStruct((_B, _S), jnp.float32),
                   jax.ShapeDtypeStruct((_B, _S), jnp.float32)),
    )(X, Y, Z)


def _knn_body(q_ref, x_ref, idx_ref, dists_ref):
    q = q_ref[0]
    xt = x_ref[0]
    dot = lax.dot_general(q, xt, (((1,), (0,)), ((), ())),
                          preferred_element_type=jnp.float32)
    qx = q[:, 0:1]
    qy = q[:, 1:2]
    qz = q[:, 2:3]
    x = xt[0:1]
    y = xt[1:2]
    z = xt[2:3]
    q2 = qx * qx + qy * qy + qz * qz
    p2 = x * x + y * y + z * z
    dists_ref[...] = (-2.0 * dot + q2) + p2
    n_iota = lax.broadcasted_iota(jnp.int32, (_S, _N), 1)
    k_iota = lax.broadcasted_iota(jnp.int32, (_S, _K), 1)

    def body(k, idxbuf):
        dcur = dists_ref[...]
        m = jnp.min(dcur, axis=1, keepdims=True)
        am = jnp.min(jnp.where(dcur == m, n_iota, jnp.int32(_N)),
                     axis=1, keepdims=True)
        idxbuf = jnp.where(k_iota == k, am, idxbuf)
        dists_ref[...] = jnp.where(n_iota == am, jnp.float32(jnp.inf), dcur)
        return idxbuf

    idx_ref[0] = lax.fori_loop(0, _K, body,
                               jnp.zeros((_S, _K), jnp.int32))


def _knn(newq8, xt8):
    return pl.pallas_call(
        _knn_body,
        grid=(_B,),
        in_specs=[pl.BlockSpec((1, _S, 8), lambda b: (b, 0, 0)),
                  pl.BlockSpec((1, 8, _N), lambda b: (b, 0, 0))],
        out_specs=pl.BlockSpec((1, _S, _K), lambda b: (b, 0, 0)),
        out_shape=jax.ShapeDtypeStruct((_B, _S, _K), jnp.int32),
        scratch_shapes=[pltpu.VMEM((_S, _N), jnp.float32)],
    )(newq8, xt8)


def _sc_group(points_flat, nbr_idx, ctr_idx):
    info = plsc.get_sparse_core_info()
    nc, ns = info.num_cores, info.num_subcores
    nw = nc * ns
    pw = (_B * _S) // nw

    mesh = plsc.VectorSubcoreMesh(core_axis_name="c", subcore_axis_name="s")

    @functools.partial(
        pl.kernel,
        mesh=mesh,
        out_type=jax.ShapeDtypeStruct((_B * _S * _K, 2 * _D), jnp.float32),
        scratch_types=[
            pltpu.VMEM((pw,), jnp.int32),
            pltpu.VMEM((pw, _D), jnp.float32),
            pltpu.VMEM((pw, _K), jnp.int32),
            pltpu.VMEM((_K, _D), jnp.float32),
            pltpu.VMEM((_K, _D), jnp.float32),
            pltpu.VMEM((_K, 2 * _D), jnp.float32),
            pltpu.VMEM((_K, 2 * _D), jnp.float32),
            pltpu.SemaphoreType.DMA,
            pltpu.SemaphoreType.DMA,
            pltpu.SemaphoreType.DMA,
            pltpu.SemaphoreType.DMA,
        ],
    )
    def grouper(points_hbm, nbr_hbm, ctr_hbm, out_hbm,
                cidx_v, crows_v, nidx_v, n0, n1, o0, o1, gs0, gs1, os0, os1):
        wid = lax.axis_index("s") * nc + lax.axis_index("c")
        base = wid * pw
        pltpu.sync_copy(ctr_hbm.at[pl.ds(base, pw)], cidx_v)
        pltpu.async_copy(points_hbm.at[cidx_v], crows_v, gs0).wait()
        pltpu.sync_copy(nbr_hbm.at[pl.ds(base, pw)], nidx_v)

        def compute(p, nrows_v, otile_v):
            cvecs = [crows_v[p, pl.ds(16 * j, 16)] for j in range(_D // 16)]

            def k_body(k, __):
                for j in range(_D // 16):
                    v = nrows_v[k, pl.ds(16 * j, 16)]
                    otile_v[k, pl.ds(16 * j, 16)] = v - cvecs[j]
                    otile_v[k, pl.ds(_D + 16 * j, 16)] = cvecs[j]
                return 0

            lax.fori_loop(0, _K, k_body, 0)

        pltpu.async_copy(points_hbm.at[nidx_v.at[0]], n0, gs0)

        def outer(i, _):
            g = 2 * i
            pltpu.make_async_copy(points_hbm.at[nidx_v.at[g]], n0, gs0).wait()
            pltpu.async_copy(points_hbm.at[nidx_v.at[g + 1]], n1, gs1)

            @pl.when(g > 0)
            def _():
                pltpu.make_async_copy(
                    o0, out_hbm.at[pl.ds((base + g - 2) * _K, _K)], os0).wait()

            compute(g, n0, o0)
            pltpu.async_copy(o0, out_hbm.at[pl.ds((base + g) * _K, _K)], os0)

            pltpu.make_async_copy(points_hbm.at[nidx_v.at[g + 1]], n1, gs1).wait()

            @pl.when(g < pw - 2)
            def _():
                pltpu.async_copy(points_hbm.at[nidx_v.at[g + 2]], n0, gs0)

            @pl.when(g > 0)
            def _():
                pltpu.make_async_copy(
                    o1, out_hbm.at[pl.ds((base + g - 1) * _K, _K)], os1).wait()

            compute(g + 1, n1, o1)
            pltpu.async_copy(o1, out_hbm.at[pl.ds((base + g + 1) * _K, _K)], os1)
            return 0

        lax.fori_loop(0, pw // 2, outer, 0)
        pltpu.make_async_copy(
            o0, out_hbm.at[pl.ds((base + pw - 2) * _K, _K)], os0).wait()
        pltpu.make_async_copy(
            o1, out_hbm.at[pl.ds((base + pw - 1) * _K, _K)], os1).wait()

    return grouper(points_flat, nbr_idx, ctr_idx)


def kernel(xyz, points):
    X = xyz[:, :, 0]
    Y = xyz[:, :, 1]
    Z = xyz[:, :, 2]
    Xr = X.reshape(_B, _ROWS, _COLS)
    Yr = Y.reshape(_B, _ROWS, _COLS)
    Zr = Z.reshape(_B, _ROWS, _COLS)

    fps_idx, qx, qy, qz = _fps(Xr, Yr, Zr)
    new_xyz = jnp.stack([qx, qy, qz], axis=-1)

    zq = jnp.zeros_like(qx)
    newq8 = jnp.stack([qx, qy, qz, zq, zq, zq, zq, zq], axis=-1)
    zp = jnp.zeros_like(X)
    xt8 = jnp.stack([X, Y, Z, zp, zp, zp, zp, zp], axis=1)
    idx = _knn(newq8, xt8)

    offs = jnp.arange(_B, dtype=jnp.int32) * _N
    ctr_flat = (fps_idx + offs[:, None]).reshape(_B * _S)
    nbr_flat = (idx + offs[:, None, None]).reshape(_B * _S, _K)
    out_flat = _sc_group(points.reshape(_B * _N, _D), nbr_flat, ctr_flat)
    new_points_out = out_flat.reshape(_B, _S, _K, 2 * _D)
    return new_xyz, new_points_out


# speedup vs baseline: 1.1898x; 1.1898x over previous
import functools

import jax
import jax.numpy as jnp
from jax import lax
from jax.experimental import pallas as pl
from jax.experimental.pallas import tpu as pltpu
from jax.experimental.pallas import tpu_sc as plsc

_B = 4
_N = 2048
_S = 512
_K = 32
_D = 256
_ROWS = 8
_COLS = _N // _ROWS


def _fps_body(x_ref, y_ref, z_ref, idx_ref, qx_ref, qy_ref, qz_ref):
    X = x_ref[...]
    Y = y_ref[...]
    Z = z_ref[...]
    pos = (lax.broadcasted_iota(jnp.int32, (_B, _ROWS, _COLS), 1) * _COLS
           + lax.broadcasted_iota(jnp.int32, (_B, _ROWS, _COLS), 2))
    cols = lax.broadcasted_iota(jnp.int32, (_B, _S), 1)

    def red(op, a):
        return op(op(a, axis=2, keepdims=True), axis=1, keepdims=True)

    def body(i, carry):
        dists, far, idx_acc, qxa, qya, qza = carry
        sel = pos == far
        cx = red(jnp.sum, jnp.where(sel, X, 0.0))
        cy = red(jnp.sum, jnp.where(sel, Y, 0.0))
        cz = red(jnp.sum, jnp.where(sel, Z, 0.0))
        dx = X - cx
        dy = Y - cy
        dz = Z - cz
        d = dx * dx + dy * dy + dz * dz
        dists = jnp.minimum(dists, d)
        wm = cols == i
        idx_acc = jnp.where(wm, far.reshape(_B, 1), idx_acc)
        qxa = jnp.where(wm, cx.reshape(_B, 1), qxa)
        qya = jnp.where(wm, cy.reshape(_B, 1), qya)
        qza = jnp.where(wm, cz.reshape(_B, 1), qza)
        m = red(jnp.max, dists)
        far = red(jnp.min, jnp.where(dists == m, pos, jnp.int32(_N)))
        return dists, far, idx_acc, qxa, qya, qza

    init = (jnp.full((_B, _ROWS, _COLS), 1e10, jnp.float32),
            jnp.zeros((_B, 1, 1), jnp.int32),
            jnp.zeros((_B, _S), jnp.int32),
            jnp.zeros((_B, _S), jnp.float32),
            jnp.zeros((_B, _S), jnp.float32),
            jnp.zeros((_B, _S), jnp.float32))
    _, _, idx_acc, qxa, qya, qza = lax.fori_loop(0, _S, body, init)
    idx_ref[...] = idx_acc
    qx_ref[...] = qxa
    qy_ref[...] = qya
    qz_ref[...] = qza


def _fps(X, Y, Z):
    return pl.pallas_call(
        _fps_body,
        out_shape=(jax.ShapeDtypeStruct((_B, _S), jnp.int32),
                   jax.ShapeDtypeStruct((_B, _S), jnp.float32),
                   jax.ShapeDtypeStruct((_B, _S), jnp.float32),
                   jax.ShapeDtypeStruct((_B, _S), jnp.float32)),
    )(X, Y, Z)


def _knn_body(q_ref, x_ref, idx_ref, dists_ref):
    q = q_ref[0]
    xt = x_ref[0]
    dot = lax.dot_general(q, xt, (((1,), (0,)), ((), ())),
                          preferred_element_type=jnp.float32)
    qx = q[:, 0:1]
    qy = q[:, 1:2]
    qz = q[:, 2:3]
    x = xt[0:1]
    y = xt[1:2]
    z = xt[2:3]
    q2 = qx * qx + qy * qy + qz * qz
    p2 = x * x + y * y + z * z
    dists_ref[...] = (-2.0 * dot + q2) + p2
    n_iota = lax.broadcasted_iota(jnp.int32, (_S, _N), 1)
    k_iota = lax.broadcasted_iota(jnp.int32, (_S, _K), 1)

    def body(k, carry):
        idxbuf, am_prev = carry
        dmod = jnp.where(n_iota == am_prev, jnp.float32(jnp.inf),
                         dists_ref[...])
        dists_ref[...] = dmod
        m = jnp.min(dmod, axis=1, keepdims=True)
        am = jnp.min(jnp.where(dists_ref[...] == m, n_iota, jnp.int32(_N)),
                     axis=1, keepdims=True)
        idxbuf = jnp.where(k_iota == k, am, idxbuf)
        return idxbuf, am

    idxbuf0 = jnp.zeros((_S, _K), jnp.int32)
    am0 = jnp.full((_S, 1), _N, jnp.int32)
    idx_ref[0] = lax.fori_loop(0, _K, body, (idxbuf0, am0))[0]


def _knn(newq8, xt8):
    return pl.pallas_call(
        _knn_body,
        grid=(_B,),
        in_specs=[pl.BlockSpec((1, _S, 8), lambda b: (b, 0, 0)),
                  pl.BlockSpec((1, 8, _N), lambda b: (b, 0, 0))],
        out_specs=pl.BlockSpec((1, _S, _K), lambda b: (b, 0, 0)),
        out_shape=jax.ShapeDtypeStruct((_B, _S, _K), jnp.int32),
        scratch_shapes=[pltpu.VMEM((_S, _N), jnp.float32)],
    )(newq8, xt8)


def _sc_group(points_flat, nbr_idx, ctr_idx):
    info = plsc.get_sparse_core_info()
    nc, ns = info.num_cores, info.num_subcores
    nw = nc * ns
    pw = (_B * _S) // nw

    mesh = plsc.VectorSubcoreMesh(core_axis_name="c", subcore_axis_name="s")

    @functools.partial(
        pl.kernel,
        mesh=mesh,
        out_type=jax.ShapeDtypeStruct((_B * _S * _K, 2 * _D), jnp.float32),
        scratch_types=[
            pltpu.VMEM((pw,), jnp.int32),
            pltpu.VMEM((pw, _D), jnp.float32),
            pltpu.VMEM((pw, _K), jnp.int32),
            pltpu.VMEM((_K, _D), jnp.float32),
            pltpu.VMEM((_K, _D), jnp.float32),
            pltpu.VMEM((_K, 2 * _D), jnp.float32),
            pltpu.VMEM((_K, 2 * _D), jnp.float32),
            pltpu.SemaphoreType.DMA,
            pltpu.SemaphoreType.DMA,
            pltpu.SemaphoreType.DMA,
            pltpu.SemaphoreType.DMA,
        ],
    )
    def grouper(points_hbm, nbr_hbm, ctr_hbm, out_hbm,
                cidx_v, crows_v, nidx_v, n0, n1, o0, o1, gs0, gs1, os0, os1):
        wid = lax.axis_index("s") * nc + lax.axis_index("c")
        base = wid * pw
        pltpu.sync_copy(ctr_hbm.at[pl.ds(base, pw)], cidx_v)
        pltpu.async_copy(points_hbm.at[cidx_v], crows_v, gs0).wait()
        pltpu.sync_copy(nbr_hbm.at[pl.ds(base, pw)], nidx_v)

        def compute(p, nrows_v, otile_v):
            cvecs = [crows_v[p, pl.ds(16 * j, 16)] for j in range(_D // 16)]

            def k_body(k, __):
                for j in range(_D // 16):
                    v = nrows_v[k, pl.ds(16 * j, 16)]
                    otile_v[k, pl.ds(16 * j, 16)] = v - cvecs[j]
                    otile_v[k, pl.ds(_D + 16 * j, 16)] = cvecs[j]
                return 0

            lax.fori_loop(0, _K, k_body, 0)

        pltpu.async_copy(points_hbm.at[nidx_v.at[0]], n0, gs0)

        def outer(i, _):
            g = 2 * i
            pltpu.make_async_copy(points_hbm.at[nidx_v.at[g]], n0, gs0).wait()
            pltpu.async_copy(points_hbm.at[nidx_v.at[g + 1]], n1, gs1)

            @pl.when(g > 0)
            def _():
                pltpu.make_async_copy(
                    o0, out_hbm.at[pl.ds((base + g - 2) * _K, _K)], os0).wait()

            compute(g, n0, o0)
            pltpu.async_copy(o0, out_hbm.at[pl.ds((base + g) * _K, _K)], os0)

            pltpu.make_async_copy(points_hbm.at[nidx_v.at[g + 1]], n1, gs1).wait()

            @pl.when(g < pw - 2)
            def _():
                pltpu.async_copy(points_hbm.at[nidx_v.at[g + 2]], n0, gs0)

            @pl.when(g > 0)
            def _():
                pltpu.make_async_copy(
                    o1, out_hbm.at[pl.ds((base + g - 1) * _K, _K)], os1).wait()

            compute(g + 1, n1, o1)
            pltpu.async_copy(o1, out_hbm.at[pl.ds((base + g + 1) * _K, _K)], os1)
            return 0

        lax.fori_loop(0, pw // 2, outer, 0)
        pltpu.make_async_copy(
            o0, out_hbm.at[pl.ds((base + pw - 2) * _K, _K)], os0).wait()
        pltpu.make_async_copy(
            o1, out_hbm.at[pl.ds((base + pw - 1) * _K, _K)], os1).wait()

    return grouper(points_flat, nbr_idx, ctr_idx)


def kernel(xyz, points):
    X = xyz[:, :, 0]
    Y = xyz[:, :, 1]
    Z = xyz[:, :, 2]
    Xr = X.reshape(_B, _ROWS, _COLS)
    Yr = Y.reshape(_B, _ROWS, _COLS)
    Zr = Z.reshape(_B, _ROWS, _COLS)

    fps_idx, qx, qy, qz = _fps(Xr, Yr, Zr)
    new_xyz = jnp.stack([qx, qy, qz], axis=-1)

    zq = jnp.zeros_like(qx)
    newq8 = jnp.stack([qx, qy, qz, zq, zq, zq, zq, zq], axis=-1)
    zp = jnp.zeros_like(X)
    xt8 = jnp.stack([X, Y, Z, zp, zp, zp, zp, zp], axis=1)
    idx = _knn(newq8, xt8)

    offs = jnp.arange(_B, dtype=jnp.int32) * _N
    ctr_flat = (fps_idx + offs[:, None]).reshape(_B * _S)
    nbr_flat = (idx + offs[:, None, None]).reshape(_B * _S, _K)
    out_flat = _sc_group(points.reshape(_B * _N, _D), nbr_flat, ctr_flat)
    new_points_out = out_flat.reshape(_B, _S, _K, 2 * _D)
    return new_xyz, new_points_out


# speedup vs baseline: 1.2487x; 1.0495x over previous
import functools

import jax
import jax.numpy as jnp
from jax import lax
from jax.experimental import pallas as pl
from jax.experimental.pallas import tpu as pltpu
from jax.experimental.pallas import tpu_sc as plsc

_B = 4
_N = 2048
_S = 512
_K = 32
_D = 256
_ROWS = 8
_COLS = _N // _ROWS


def _fps_body(x_ref, y_ref, z_ref, idx_ref, qx_ref, qy_ref, qz_ref):
    X = x_ref[...]
    Y = y_ref[...]
    Z = z_ref[...]
    pos = (lax.broadcasted_iota(jnp.int32, (_B, _ROWS, _COLS), 1) * _COLS
           + lax.broadcasted_iota(jnp.int32, (_B, _ROWS, _COLS), 2))
    cols = lax.broadcasted_iota(jnp.int32, (_B, _S), 1)

    def red(op, a):
        return op(op(a, axis=2, keepdims=True), axis=1, keepdims=True)

    def body(i, carry):
        dists, far, idx_acc, qxa, qya, qza = carry
        sel = pos == far
        cx = red(jnp.sum, jnp.where(sel, X, 0.0))
        cy = red(jnp.sum, jnp.where(sel, Y, 0.0))
        cz = red(jnp.sum, jnp.where(sel, Z, 0.0))
        dx = X - cx
        dy = Y - cy
        dz = Z - cz
        d = dx * dx + dy * dy + dz * dz
        dists = jnp.minimum(dists, d)
        wm = cols == i
        idx_acc = jnp.where(wm, far.reshape(_B, 1), idx_acc)
        qxa = jnp.where(wm, cx.reshape(_B, 1), qxa)
        qya = jnp.where(wm, cy.reshape(_B, 1), qya)
        qza = jnp.where(wm, cz.reshape(_B, 1), qza)
        m = red(jnp.max, dists)
        far = red(jnp.min, jnp.where(dists == m, pos, jnp.int32(_N)))
        return dists, far, idx_acc, qxa, qya, qza

    init = (jnp.full((_B, _ROWS, _COLS), 1e10, jnp.float32),
            jnp.zeros((_B, 1, 1), jnp.int32),
            jnp.zeros((_B, _S), jnp.int32),
            jnp.zeros((_B, _S), jnp.float32),
            jnp.zeros((_B, _S), jnp.float32),
            jnp.zeros((_B, _S), jnp.float32))
    _, _, idx_acc, qxa, qya, qza = lax.fori_loop(0, _S, body, init)
    idx_ref[...] = idx_acc
    qx_ref[...] = qxa
    qy_ref[...] = qya
    qz_ref[...] = qza


def _fps(X, Y, Z):
    return pl.pallas_call(
        _fps_body,
        out_shape=(jax.ShapeDtypeStruct((_B, _S), jnp.int32),
                   jax.ShapeDtypeStruct((_B, _S), jnp.float32),
                   jax.ShapeDtypeStruct((_B, _S), jnp.float32),
                   jax.ShapeDtypeStruct((_B, _S), jnp.float32)),
    )(X, Y, Z)


def _knn_body(q_ref, x_ref, idx_ref, dists_ref):
    q = q_ref[0]
    xt = x_ref[0]
    dot = lax.dot_general(q, xt, (((1,), (0,)), ((), ())),
                          preferred_element_type=jnp.float32)
    qx = q[:, 0:1]
    qy = q[:, 1:2]
    qz = q[:, 2:3]
    x = xt[0:1]
    y = xt[1:2]
    z = xt[2:3]
    q2 = qx * qx + qy * qy + qz * qz
    p2 = x * x + y * y + z * z
    dists_ref[...] = (-2.0 * dot + q2) + p2
    n_iota = lax.broadcasted_iota(jnp.int32, (_S, _N), 1)
    k_iota = lax.broadcasted_iota(jnp.int32, (_S, _K), 1)

    def body(k, idxbuf):
        dcur = dists_ref[...]
        m = jnp.min(dcur, axis=1, keepdims=True)
        am = jnp.min(jnp.where(dcur == m, n_iota, jnp.int32(_N)),
                     axis=1, keepdims=True)
        idxbuf = jnp.where(k_iota == k, am, idxbuf)
        dists_ref[...] = jnp.where(n_iota == am, jnp.float32(jnp.inf), dcur)
        return idxbuf

    idx_ref[0] = lax.fori_loop(0, _K, body,
                               jnp.zeros((_S, _K), jnp.int32))


def _knn(newq8, xt8):
    return pl.pallas_call(
        _knn_body,
        grid=(_B,),
        in_specs=[pl.BlockSpec((1, _S, 8), lambda b: (b, 0, 0)),
                  pl.BlockSpec((1, 8, _N), lambda b: (b, 0, 0))],
        out_specs=pl.BlockSpec((1, _S, _K), lambda b: (b, 0, 0)),
        out_shape=jax.ShapeDtypeStruct((_B, _S, _K), jnp.int32),
        scratch_shapes=[pltpu.VMEM((_S, _N), jnp.float32)],
    )(newq8, xt8)


def _sc_group(points_flat, nbr_idx, ctr_idx):
    info = plsc.get_sparse_core_info()
    nc, ns = info.num_cores, info.num_subcores
    nw = nc * ns
    pw = (_B * _S) // nw

    mesh = plsc.VectorSubcoreMesh(core_axis_name="c", subcore_axis_name="s")

    @functools.partial(
        pl.kernel,
        mesh=mesh,
        out_type=jax.ShapeDtypeStruct((_B * _S * _K, 2 * _D), jnp.float32),
        scratch_types=[
            pltpu.VMEM((pw,), jnp.int32),
            pltpu.VMEM((pw, _D), jnp.float32),
            pltpu.VMEM((pw, _K), jnp.int32),
            pltpu.VMEM((_K, _D), jnp.float32),
            pltpu.VMEM((_K, _D), jnp.float32),
            pltpu.VMEM((_K, 2 * _D), jnp.float32),
            pltpu.VMEM((_K, 2 * _D), jnp.float32),
            pltpu.SemaphoreType.DMA,
            pltpu.SemaphoreType.DMA,
            pltpu.SemaphoreType.DMA,
            pltpu.SemaphoreType.DMA,
        ],
    )
    def grouper(points_hbm, nbr_hbm, ctr_hbm, out_hbm,
                cidx_v, crows_v, nidx_v, n0, n1, o0, o1, gs0, gs1, os0, os1):
        wid = lax.axis_index("s") * nc + lax.axis_index("c")
        base = wid * pw
        pltpu.sync_copy(ctr_hbm.at[pl.ds(base, pw)], cidx_v)
        pltpu.async_copy(points_hbm.at[cidx_v], crows_v, gs0).wait()
        pltpu.sync_copy(nbr_hbm.at[pl.ds(base, pw)], nidx_v)

        def compute(p, nrows_v, otile_v):
            cvecs = [crows_v[p, pl.ds(16 * j, 16)] for j in range(_D // 16)]

            def k_body(k, __):
                for j in range(_D // 16):
                    v = nrows_v[k, pl.ds(16 * j, 16)]
                    otile_v[k, pl.ds(16 * j, 16)] = v - cvecs[j]
                    otile_v[k, pl.ds(_D + 16 * j, 16)] = cvecs[j]
                return 0

            lax.fori_loop(0, _K, k_body, 0)

        pltpu.async_copy(points_hbm.at[nidx_v.at[0]], n0, gs0)

        def outer(i, _):
            g = 2 * i
            pltpu.make_async_copy(points_hbm.at[nidx_v.at[g]], n0, gs0).wait()
            pltpu.async_copy(points_hbm.at[nidx_v.at[g + 1]], n1, gs1)

            @pl.when(g > 0)
            def _():
                pltpu.make_async_copy(
                    o0, out_hbm.at[pl.ds((base + g - 2) * _K, _K)], os0).wait()

            compute(g, n0, o0)
            pltpu.async_copy(o0, out_hbm.at[pl.ds((base + g) * _K, _K)], os0)

            pltpu.make_async_copy(points_hbm.at[nidx_v.at[g + 1]], n1, gs1).wait()

            @pl.when(g < pw - 2)
            def _():
                pltpu.async_copy(points_hbm.at[nidx_v.at[g + 2]], n0, gs0)

            @pl.when(g > 0)
            def _():
                pltpu.make_async_copy(
                    o1, out_hbm.at[pl.ds((base + g - 1) * _K, _K)], os1).wait()

            compute(g + 1, n1, o1)
            pltpu.async_copy(o1, out_hbm.at[pl.ds((base + g + 1) * _K, _K)], os1)
            return 0

        lax.fori_loop(0, pw // 2, outer, 0)
        pltpu.make_async_copy(
            o0, out_hbm.at[pl.ds((base + pw - 2) * _K, _K)], os0).wait()
        pltpu.make_async_copy(
            o1, out_hbm.at[pl.ds((base + pw - 1) * _K, _K)], os1).wait()

    return grouper(points_flat, nbr_idx, ctr_idx)


def kernel(xyz, points):
    X = xyz[:, :, 0]
    Y = xyz[:, :, 1]
    Z = xyz[:, :, 2]
    Xr = X.reshape(_B, _ROWS, _COLS)
    Yr = Y.reshape(_B, _ROWS, _COLS)
    Zr = Z.reshape(_B, _ROWS, _COLS)

    fps_idx, qx, qy, qz = _fps(Xr, Yr, Zr)
    new_xyz = jnp.stack([qx, qy, qz], axis=-1)

    zq = jnp.zeros_like(qx)
    newq8 = jnp.stack([qx, qy, qz, zq, zq, zq, zq, zq], axis=-1)
    zp = jnp.zeros_like(X)
    xt8 = jnp.stack([X, Y, Z, zp, zp, zp, zp, zp], axis=1)
    idx = _knn(newq8, xt8)

    offs = jnp.arange(_B, dtype=jnp.int32) * _N
    ctr_flat = (fps_idx + offs[:, None]).reshape(_B * _S)
    nbr_flat = (idx + offs[:, None, None]).reshape(_B * _S, _K)
    out_flat = _sc_group(points.reshape(_B * _N, _D), nbr_flat, ctr_flat)
    new_points_out = out_flat.reshape(_B, _S, _K, 2 * _D)
    return new_xyz, new_points_out
